# Initial kernel scaffold; baseline (speedup 1.0000x reference)
import jax
import jax.numpy as jnp
from jax.experimental import pallas as pl


def kernel(x, embeddings):
    raise NotImplementedError("write your pallas kernel here")


# speedup vs baseline: 1.0071x; 1.0071x over previous
import functools

import jax
import jax.numpy as jnp
from jax import lax
from jax.experimental import pallas as pl
from jax.experimental.pallas import tpu as pltpu
from jax.experimental.pallas import tpu_sc as plsc

_NUM_EMBEDDINGS = 8192
_DIM = 256
_ROWS = 4096

_RB = 1024
_CB = 1024


def _argmin_body(xb, eb, xn, en, out, minv, mini):
    c = pl.program_id(1)

    @pl.when(c == 0)
    def _init():
        minv[...] = jnp.full((_RB, 1), jnp.inf, dtype=jnp.float32)
        mini[...] = jnp.zeros((_RB, 1), dtype=jnp.int32)

    sim = jnp.dot(xb[...], eb[...], preferred_element_type=jnp.float32)
    d = (xn[...] + en[...]) - 2.0 * sim
    lmin = jnp.min(d, axis=1, keepdims=True)
    gcol = c * _CB + lax.broadcasted_iota(jnp.int32, (_RB, _CB), 1)
    lidx = jnp.min(
        jnp.where(d == lmin, gcol, jnp.int32(2**30)), axis=1, keepdims=True
    )
    better = lmin < minv[...]
    mini[...] = jnp.where(better, lidx, mini[...])
    minv[...] = jnp.where(better, lmin, minv[...])

    @pl.when(c == pl.num_programs(1) - 1)
    def _write():
        out[...] = mini[...].reshape(1, 1, _RB)


def _tc_argmin(flattened, embeddings, x_norm, e_norm):
    nr = _ROWS // _RB
    nc = _NUM_EMBEDDINGS // _CB
    out = pl.pallas_call(
        _argmin_body,
        grid=(nr, nc),
        in_specs=[
            pl.BlockSpec((_RB, _DIM), lambda r, c: (r, 0)),
            pl.BlockSpec((_DIM, _CB), lambda r, c: (0, c)),
            pl.BlockSpec((_RB, 1), lambda r, c: (r, 0)),
            pl.BlockSpec((1, _CB), lambda r, c: (0, c)),
        ],
        out_specs=pl.BlockSpec((1, 1, _RB), lambda r, c: (r, 0, 0)),
        out_shape=jax.ShapeDtypeStruct((nr, 1, _RB), jnp.int32),
        scratch_shapes=[
            pltpu.VMEM((_RB, 1), jnp.float32),
            pltpu.VMEM((_RB, 1), jnp.int32),
        ],
    )(flattened, embeddings, x_norm, e_norm)
    return out.reshape(_ROWS)


def _sc_gather(table, idx):
    info = plsc.get_sparse_core_info()
    ncores, nsub = info.num_cores, info.num_subcores
    nw = ncores * nsub
    b_per_w = _ROWS // nw
    mesh = plsc.VectorSubcoreMesh(core_axis_name="c", subcore_axis_name="s")

    @functools.partial(
        pl.kernel,
        mesh=mesh,
        out_type=jax.ShapeDtypeStruct((_ROWS, _DIM), jnp.float32),
        scratch_types=[
            pltpu.VMEM((b_per_w,), jnp.int32),
            pltpu.VMEM((b_per_w, _DIM), jnp.float32),
            pltpu.SemaphoreType.DMA,
        ],
    )
    def gk(table_hbm, idx_hbm, out_hbm, idx_v, rows_v, sem):
        wid = lax.axis_index("s") * ncores + lax.axis_index("c")
        base = wid * b_per_w
        pltpu.sync_copy(idx_hbm.at[pl.ds(base, b_per_w)], idx_v)
        pltpu.async_copy(table_hbm.at[idx_v], rows_v, sem).wait()
        pltpu.sync_copy(rows_v, out_hbm.at[pl.ds(base, b_per_w)])

    return gk(table, idx)


def kernel(x, embeddings):
    input_shape = x.shape
    flattened = jnp.reshape(x, (-1, _DIM))
    x_norm = jnp.sum(flattened**2, axis=1, keepdims=True)
    e_norm = jnp.reshape(jnp.sum(embeddings**2, axis=0), (1, _NUM_EMBEDDINGS))

    idx = _tc_argmin(flattened, embeddings, x_norm, e_norm)

    quantized = _sc_gather(embeddings.T, idx)
    quantized = jnp.reshape(quantized, input_shape)

    quantized_st = x + lax.stop_gradient(quantized - x)
    return (quantized_st, idx)


# speedup vs baseline: 1.1745x; 1.1662x over previous
import functools

import jax
import jax.numpy as jnp
from jax import lax
from jax.experimental import pallas as pl
from jax.experimental.pallas import tpu as pltpu
from jax.experimental.pallas import tpu_sc as plsc

_NUM_EMBEDDINGS = 8192
_DIM = 256
_ROWS = 4096

_RB = 1024
_CB = 1024


def _argmin_body(xb, e2b, xn, en, out, minv, mini):
    c = pl.program_id(1)

    @pl.when(c == 0)
    def _init():
        minv[...] = jnp.full((_RB, 128), jnp.inf, dtype=jnp.float32)
        mini[...] = jnp.zeros((_RB, 128), dtype=jnp.int32)

    s2 = jnp.dot(xb[...], e2b[...], preferred_element_type=jnp.float32)
    xn_v = xn[...]
    en_v = en[...]
    rm = minv[...]
    ri = mini[...]
    lane = lax.broadcasted_iota(jnp.int32, (1, 128), 1)
    for k in range(_CB // 128):
        sl = slice(k * 128, (k + 1) * 128)
        dk = (xn_v + en_v[:, sl]) + s2[:, sl]
        ik = jnp.broadcast_to(c * _CB + k * 128 + lane, (_RB, 128))
        upd = dk < rm
        rm = jnp.where(upd, dk, rm)
        ri = jnp.where(upd, ik, ri)
    minv[...] = rm
    mini[...] = ri

    @pl.when(c == pl.num_programs(1) - 1)
    def _write():
        m = jnp.min(rm, axis=1, keepdims=True)
        cand = jnp.where(rm == m, ri, jnp.int32(2**30))
        out[...] = jnp.min(cand, axis=1).reshape(1, 1, _RB)


def _tc_argmin(flattened, emb_neg2, x_norm, e_norm):
    nr = _ROWS // _RB
    nc = _NUM_EMBEDDINGS // _CB
    out = pl.pallas_call(
        _argmin_body,
        grid=(nr, nc),
        in_specs=[
            pl.BlockSpec((_RB, _DIM), lambda r, c: (r, 0)),
            pl.BlockSpec((_DIM, _CB), lambda r, c: (0, c)),
            pl.BlockSpec((_RB, 1), lambda r, c: (r, 0)),
            pl.BlockSpec((1, _CB), lambda r, c: (0, c)),
        ],
        out_specs=pl.BlockSpec((1, 1, _RB), lambda r, c: (r, 0, 0)),
        out_shape=jax.ShapeDtypeStruct((nr, 1, _RB), jnp.int32),
        scratch_shapes=[
            pltpu.VMEM((_RB, 128), jnp.float32),
            pltpu.VMEM((_RB, 128), jnp.int32),
        ],
    )(flattened, emb_neg2, x_norm, e_norm)
    return out.reshape(_ROWS)


def _sc_gather(table, idx):
    info = plsc.get_sparse_core_info()
    ncores, nsub = info.num_cores, info.num_subcores
    nw = ncores * nsub
    b_per_w = _ROWS // nw
    mesh = plsc.VectorSubcoreMesh(core_axis_name="c", subcore_axis_name="s")

    @functools.partial(
        pl.kernel,
        mesh=mesh,
        out_type=jax.ShapeDtypeStruct((_ROWS, _DIM), jnp.float32),
        scratch_types=[
            pltpu.VMEM((b_per_w,), jnp.int32),
            pltpu.VMEM((b_per_w, _DIM), jnp.float32),
            pltpu.SemaphoreType.DMA,
        ],
    )
    def gk(table_hbm, idx_hbm, out_hbm, idx_v, rows_v, sem):
        wid = lax.axis_index("s") * ncores + lax.axis_index("c")
        base = wid * b_per_w
        pltpu.sync_copy(idx_hbm.at[pl.ds(base, b_per_w)], idx_v)
        pltpu.async_copy(table_hbm.at[idx_v], rows_v, sem).wait()
        pltpu.sync_copy(rows_v, out_hbm.at[pl.ds(base, b_per_w)])

    return gk(table, idx)


def kernel(x, embeddings):
    input_shape = x.shape
    flattened = jnp.reshape(x, (-1, _DIM))
    x_norm = jnp.sum(flattened**2, axis=1, keepdims=True)
    e_norm = jnp.reshape(jnp.sum(embeddings**2, axis=0), (1, _NUM_EMBEDDINGS))

    idx = _tc_argmin(flattened, -2.0 * embeddings, x_norm, e_norm)

    quantized = _sc_gather(embeddings.T, idx)
    quantized = jnp.reshape(quantized, input_shape)

    quantized_st = x + lax.stop_gradient(quantized - x)
    return (quantized_st, idx)
